# baseline (speedup 1.0000x reference)
import jax
import jax.numpy as jnp
from jax import lax
from jax.experimental import pallas as pl
from jax.experimental.pallas import tpu as pltpu
from jax.experimental.pallas import tpu_sc as plsc

NB_CLS = 100000
NB_REL = 1000
EMB = 128
HD = EMB // 2
D = EMB + 1
B = 4096
NC = 2
NS = 16
NW = NC * NS
BPW = B // NW
CH = 32
NQ = BPW // CH
NT = 4
MARGIN = 0.01
INF = 5.0

CB = 4096
QB = CB // 4
NBLK = -(-NB_CLS // CB)
NRAD = NBLK * QB
MASKHI = -65536


def _pack_dims(xf):
    u = lax.bitcast_convert_type(xf, jnp.uint32)
    h = (u + jnp.uint32(0x8000)) >> 16
    h3 = h.reshape(HD, 2, xf.shape[1])
    return h3[:, 0, :] | (h3[:, 1, :] << 16)


def _prep_body(xt_ref, cx_ref, rad_ref):
    xb = xt_ref[...]
    w = _pack_dims(xb[:EMB, :])
    wt = lax.bitcast_convert_type(jnp.transpose(w), jnp.int32)
    wt3 = wt.reshape(CB // 2, 2, HD)
    cx_ref[:, :HD] = wt3[:, 0, :]
    cx_ref[:, HD:] = wt3[:, 1, :]
    q = (jnp.abs(xb[EMB, :]) * 255.0 + 0.5).astype(jnp.int32)
    q4 = q.reshape(4, QB)
    rad_ref[...] = (q4[0] | (q4[1] << 8) | (q4[2] << 16) | (q4[3] << 24))


def _prep_call(cls_t):
    return pl.pallas_call(
        _prep_body,
        grid=(NBLK,),
        in_specs=[pl.BlockSpec((D, CB), lambda i: (0, i))],
        out_specs=[pl.BlockSpec((CB // 2, EMB), lambda i: (i, 0)),
                   pl.BlockSpec((QB,), lambda i: (i,))],
        out_shape=[jax.ShapeDtypeStruct((NB_CLS // 2, EMB), jnp.int32),
                   jax.ShapeDtypeStruct((NRAD,), jnp.int32)],
        compiler_params=pltpu.CompilerParams(
            dimension_semantics=("arbitrary",)),
    )(cls_t)


def _prep_rel_body(xt_ref, rx_ref):
    rx_ref[...] = lax.bitcast_convert_type(
        jnp.transpose(_pack_dims(xt_ref[...])), jnp.int32)


def _prep_rel_call(rel_t):
    return pl.pallas_call(
        _prep_rel_body,
        out_shape=jax.ShapeDtypeStruct((NB_REL, HD), jnp.int32),
    )(rel_t)


def _sqrt16(x):
    xs = jnp.maximum(x, 1e-30)
    i = plsc.bitcast(xs, jnp.int32)
    y = plsc.bitcast(jnp.int32(0x5F3759DF) - (i >> 1), jnp.float32)
    for _ in range(4):
        y = y * (1.5 - 0.5 * xs * y * y)
    return xs * y


def _unpack(w):
    lo = plsc.bitcast(w << 16, jnp.float32)
    hi = plsc.bitcast(w & MASKHI, jnp.float32)
    return lo, hi


def _rad_lookup(radtab, cvec):
    off = cvec & jnp.int32(CB - 1)
    widx = ((cvec >> 12) << 10) + (off & jnp.int32(QB - 1))
    w = plsc.load_gather(radtab, [widx])
    sh = (off >> 10) << 3
    q = (w >> sh) & jnp.int32(255)
    return q.astype(jnp.float32) * (1.0 / 255.0)


def _sc_body(clsx_hbm, rad_hbm, rel_hbm, ci_hbm, di_hbm, ri_hbm, ti_hbm,
             out_hbm,
             radtab, reltab, cb0, cb1, cb2, db0, db1, db2,
             icp0, icp1, icp2, idp0, idp1, idp2,
             ica, ida, ira, tix, ob, semr, sem0, sem1, sem2):
    wid = lax.axis_index("s") * NC + lax.axis_index("c")
    base = pl.multiple_of(wid * BPW, BPW)
    ibase = pl.multiple_of(wid * (NT * BPW), NT * BPW)
    iota16 = lax.iota(jnp.int32, 16)

    rad_cp = pltpu.async_copy(rad_hbm, radtab, semr)
    rel_cp = pltpu.async_copy(rel_hbm, reltab, semr)

    pltpu.sync_copy(ci_hbm.at[pl.ds(ibase, NT * BPW)], ica)
    pltpu.sync_copy(di_hbm.at[pl.ds(ibase, NT * BPW)], ida)
    pltpu.sync_copy(ri_hbm.at[pl.ds(ibase, NT * BPW)], ira)
    pltpu.sync_copy(ti_hbm.at[pl.ds(base, BPW)], tix)

    bufs = [(cb0, db0, icp0, idp0, sem0), (cb1, db1, icp1, idp1, sem1),
            (cb2, db2, icp2, idp2, sem2)]

    def fire(k):
        cbb, dbb, icp, idp, sem = bufs[k % 3]
        off = pl.multiple_of(k * CH, CH)
        for h in range(CH // 16):
            sl = pl.ds(off + pl.multiple_of(h * 16, 16), 16)
            dsl = pl.ds(pl.multiple_of(h * 16, 16), 16)
            icp[dsl] = ica[sl] >> 1
            idp[dsl] = ida[sl] >> 1
        return (pltpu.async_copy(clsx_hbm.at[icp], cbb, sem),
                pltpu.async_copy(clsx_hbm.at[idp], dbb, sem))

    def compute(k):
        t, q = divmod(k, NQ)
        cbb, dbb = bufs[k % 3][:2]

        def group(g, carry):
            rows = pl.multiple_of(g * 16, 16) + iota16
            goff = pl.multiple_of(k * CH, CH) + pl.multiple_of(g * 16, 16)
            icv = ica[pl.ds(goff, 16)]
            idv = ida[pl.ds(goff, 16)]
            irv = ira[pl.ds(goff, 16)]
            ccol = (icv & jnp.int32(1)) << 6
            dcol = (idv & jnp.int32(1)) << 6

            def dstep(j, accs):
                a1, a2, a3 = accs
                cl, chh = _unpack(plsc.load_gather(cbb, [rows, ccol + j]))
                dl, dh = _unpack(plsc.load_gather(dbb, [rows, dcol + j]))
                rl, rh = _unpack(plsc.load_gather(
                    reltab, [irv, jnp.zeros((16,), jnp.int32) + j]))
                if t == 2:
                    tl = cl - rl - dl
                    th = chh - rh - dh
                else:
                    tl = cl + rl - dl
                    th = chh + rh - dh
                return (a1 + cl * cl + chh * chh,
                        a2 + dl * dl + dh * dh,
                        a3 + tl * tl + th * th)

            z = jnp.zeros((16,), jnp.float32)
            a1, a2, a3 = lax.fori_loop(0, HD, dstep, (z, z, z), unroll=8)
            rc = _rad_lookup(radtab, icv)
            rd = _rad_lookup(radtab, idv)
            n1 = _sqrt16(a1)
            n2 = _sqrt16(a2)
            e = _sqrt16(a3)
            reg = jnp.abs(n1 - 1.0) + jnp.abs(n2 - 1.0)
            if t in (0, 1):
                l = jnp.maximum(e + rc - rd - MARGIN, 0.0) + reg
            elif t == 2:
                l = jnp.maximum(e - rc - rd - MARGIN, 0.0) + reg
            else:
                l = (MARGIN - e + rc + rd) + reg
            sl = pl.ds(pl.multiple_of(q * CH, CH) + pl.multiple_of(g * 16, 16), 16)
            if t == 0:
                ob[sl] = l
            else:
                ob[sl] = ob[sl] + l
            return carry

        lax.fori_loop(0, CH // 16, group, 0)

    cps = {0: fire(0), 1: fire(1), 2: fire(2)}
    rad_cp.wait()
    rel_cp.wait()
    for k in range(NT * NQ):
        for cp in cps[k]:
            cp.wait()
        compute(k)
        if k + 3 < NT * NQ:
            cps[k + 3] = fire(k + 3)

    def topg(g, carry):
        sl = pl.ds(pl.multiple_of(g * 16, 16), 16)
        tv = _rad_lookup(radtab, tix[sl])
        ob[sl] = ob[sl] + jnp.abs(tv - INF)
        return carry

    lax.fori_loop(0, BPW // 16, topg, 0)
    pltpu.sync_copy(ob, out_hbm.at[pl.ds(base, BPW)])


def _make_call():
    mesh = plsc.VectorSubcoreMesh(core_axis_name="c", subcore_axis_name="s",
                                  num_cores=NC, num_subcores=NS)
    return pl.kernel(
        _sc_body,
        out_type=jax.ShapeDtypeStruct((B,), jnp.float32),
        mesh=mesh,
        compiler_params=pltpu.CompilerParams(use_tc_tiling_on_sc=False,
                                             needs_layout_passes=False),
        scratch_types=[
            pltpu.VMEM((NRAD,), jnp.int32),
            pltpu.VMEM((NB_REL, HD), jnp.int32),
            pltpu.VMEM((CH, EMB), jnp.int32),
            pltpu.VMEM((CH, EMB), jnp.int32),
            pltpu.VMEM((CH, EMB), jnp.int32),
            pltpu.VMEM((CH, EMB), jnp.int32),
            pltpu.VMEM((CH, EMB), jnp.int32),
            pltpu.VMEM((CH, EMB), jnp.int32),
            pltpu.VMEM((CH,), jnp.int32),
            pltpu.VMEM((CH,), jnp.int32),
            pltpu.VMEM((CH,), jnp.int32),
            pltpu.VMEM((CH,), jnp.int32),
            pltpu.VMEM((CH,), jnp.int32),
            pltpu.VMEM((CH,), jnp.int32),
            pltpu.VMEM((NT * BPW,), jnp.int32),
            pltpu.VMEM((NT * BPW,), jnp.int32),
            pltpu.VMEM((NT * BPW,), jnp.int32),
            pltpu.VMEM((BPW,), jnp.int32),
            pltpu.VMEM((BPW,), jnp.float32),
            pltpu.SemaphoreType.DMA,
            pltpu.SemaphoreType.DMA,
            pltpu.SemaphoreType.DMA,
            pltpu.SemaphoreType.DMA,
        ],
    )


def _arrange(cols):
    x = jnp.stack(cols, axis=0)
    x = x.reshape(NT, NW, BPW).swapaxes(0, 1)
    return x.reshape(-1)


def kernel(nf1, nf3, nf4, top, nf3_neg, cls_emb, rel_emb):
    ci = _arrange([nf1[:, 0], nf3[:, 0], nf4[:, 1], nf3_neg[:, 0]])
    di = _arrange([nf1[:, 2], nf3[:, 2], nf4[:, 2], nf3_neg[:, 2]])
    ri = _arrange([nf1[:, 1], nf3[:, 1], nf4[:, 0], nf3_neg[:, 1]])
    ti = top[:, 0]
    cls_p, rad_p = _prep_call(cls_emb.T)
    rel_p = _prep_rel_call(rel_emb.T)
    out = _make_call()(cls_p, rad_p, rel_p, ci, di, ri, ti)
    return out.reshape(B, 1)


# speedup vs baseline: 1.0117x; 1.0117x over previous
import jax
import jax.numpy as jnp
from jax import lax
from jax.experimental import pallas as pl
from jax.experimental.pallas import tpu as pltpu
from jax.experimental.pallas import tpu_sc as plsc

NB_CLS = 100000
NB_REL = 1000
EMB = 128
HD = EMB // 2
D = EMB + 1
B = 4096
NC = 2
NS = 16
NW = NC * NS
BPW = B // NW
CH = 64
NQ = BPW // CH
NT = 4
MARGIN = 0.01
INF = 5.0

CB = 4096
QB = CB // 4
NBLK = -(-NB_CLS // CB)
NRAD = NBLK * QB
MASKHI = -65536


def _pack_dims(xf):
    u = lax.bitcast_convert_type(xf, jnp.uint32)
    h = (u + jnp.uint32(0x8000)) >> 16
    h3 = h.reshape(HD, 2, xf.shape[1])
    return h3[:, 0, :] | (h3[:, 1, :] << 16)


def _prep_body(xt_ref, cx_ref, rad_ref):
    xb = xt_ref[...]
    w = _pack_dims(xb[:EMB, :])
    wt = lax.bitcast_convert_type(jnp.transpose(w), jnp.int32)
    wt3 = wt.reshape(CB // 2, 2, HD)
    cx_ref[:, :HD] = wt3[:, 0, :]
    cx_ref[:, HD:] = wt3[:, 1, :]
    q = (jnp.abs(xb[EMB, :]) * 255.0 + 0.5).astype(jnp.int32)
    q4 = q.reshape(4, QB)
    rad_ref[...] = (q4[0] | (q4[1] << 8) | (q4[2] << 16) | (q4[3] << 24))


def _prep_call(cls_t):
    return pl.pallas_call(
        _prep_body,
        grid=(NBLK,),
        in_specs=[pl.BlockSpec((D, CB), lambda i: (0, i))],
        out_specs=[pl.BlockSpec((CB // 2, EMB), lambda i: (i, 0)),
                   pl.BlockSpec((QB,), lambda i: (i,))],
        out_shape=[jax.ShapeDtypeStruct((NB_CLS // 2, EMB), jnp.int32),
                   jax.ShapeDtypeStruct((NRAD,), jnp.int32)],
        compiler_params=pltpu.CompilerParams(
            dimension_semantics=("arbitrary",)),
    )(cls_t)


def _prep_rel_body(xt_ref, rx_ref):
    rx_ref[...] = lax.bitcast_convert_type(
        jnp.transpose(_pack_dims(xt_ref[...])), jnp.int32)


def _prep_rel_call(rel_t):
    return pl.pallas_call(
        _prep_rel_body,
        out_shape=jax.ShapeDtypeStruct((NB_REL, HD), jnp.int32),
    )(rel_t)


def _sqrt16(x):
    xs = jnp.maximum(x, 1e-30)
    i = plsc.bitcast(xs, jnp.int32)
    y = plsc.bitcast(jnp.int32(0x5F3759DF) - (i >> 1), jnp.float32)
    for _ in range(4):
        y = y * (1.5 - 0.5 * xs * y * y)
    return xs * y


def _unpack(w):
    lo = plsc.bitcast(w << 16, jnp.float32)
    hi = plsc.bitcast(w & MASKHI, jnp.float32)
    return lo, hi


def _rad_lookup(radtab, cvec):
    off = cvec & jnp.int32(CB - 1)
    widx = ((cvec >> 12) << 10) + (off & jnp.int32(QB - 1))
    w = plsc.load_gather(radtab, [widx])
    sh = (off >> 10) << 3
    q = (w >> sh) & jnp.int32(255)
    return q.astype(jnp.float32) * (1.0 / 255.0)


def _sc_body(clsx_hbm, rad_hbm, rel_hbm, ci_hbm, di_hbm, ri_hbm, ti_hbm,
             out_hbm,
             radtab, reltab, cb0, cb1, db0, db1,
             icp0, icp1, idp0, idp1,
             ica, ida, ira, tix, ob, semr, sem0, sem1):
    wid = lax.axis_index("s") * NC + lax.axis_index("c")
    base = pl.multiple_of(wid * BPW, BPW)
    ibase = pl.multiple_of(wid * (NT * BPW), NT * BPW)
    iota16 = lax.iota(jnp.int32, 16)

    rad_cp = pltpu.async_copy(rad_hbm, radtab, semr)
    rel_cp = pltpu.async_copy(rel_hbm, reltab, semr)

    pltpu.sync_copy(ci_hbm.at[pl.ds(ibase, NT * BPW)], ica)
    pltpu.sync_copy(di_hbm.at[pl.ds(ibase, NT * BPW)], ida)
    pltpu.sync_copy(ri_hbm.at[pl.ds(ibase, NT * BPW)], ira)
    pltpu.sync_copy(ti_hbm.at[pl.ds(base, BPW)], tix)

    bufs = [(cb0, db0, icp0, idp0, sem0), (cb1, db1, icp1, idp1, sem1)]

    def fire(k):
        cbb, dbb, icp, idp, sem = bufs[k % 2]
        off = pl.multiple_of(k * CH, CH)
        for h in range(CH // 16):
            sl = pl.ds(off + pl.multiple_of(h * 16, 16), 16)
            dsl = pl.ds(pl.multiple_of(h * 16, 16), 16)
            icp[dsl] = ica[sl] >> 1
            idp[dsl] = ida[sl] >> 1
        return (pltpu.async_copy(clsx_hbm.at[icp], cbb, sem),
                pltpu.async_copy(clsx_hbm.at[idp], dbb, sem))

    def compute(k):
        t, q = divmod(k, NQ)
        cbb, dbb = bufs[k % 2][:2]

        def group(g, carry):
            rows = pl.multiple_of(g * 16, 16) + iota16
            goff = pl.multiple_of(k * CH, CH) + pl.multiple_of(g * 16, 16)
            icv = ica[pl.ds(goff, 16)]
            idv = ida[pl.ds(goff, 16)]
            irv = ira[pl.ds(goff, 16)]
            ccol = (icv & jnp.int32(1)) << 6
            dcol = (idv & jnp.int32(1)) << 6

            def dstep(j, accs):
                a1, a2, a3 = accs
                cl, chh = _unpack(plsc.load_gather(cbb, [rows, ccol + j]))
                dl, dh = _unpack(plsc.load_gather(dbb, [rows, dcol + j]))
                rl, rh = _unpack(plsc.load_gather(
                    reltab, [irv, jnp.zeros((16,), jnp.int32) + j]))
                if t == 2:
                    tl = cl - rl - dl
                    th = chh - rh - dh
                else:
                    tl = cl + rl - dl
                    th = chh + rh - dh
                return (a1 + cl * cl + chh * chh,
                        a2 + dl * dl + dh * dh,
                        a3 + tl * tl + th * th)

            z = jnp.zeros((16,), jnp.float32)
            a1, a2, a3 = lax.fori_loop(0, HD, dstep, (z, z, z), unroll=8)
            rc = _rad_lookup(radtab, icv)
            rd = _rad_lookup(radtab, idv)
            n1 = _sqrt16(a1)
            n2 = _sqrt16(a2)
            e = _sqrt16(a3)
            reg = jnp.abs(n1 - 1.0) + jnp.abs(n2 - 1.0)
            if t in (0, 1):
                l = jnp.maximum(e + rc - rd - MARGIN, 0.0) + reg
            elif t == 2:
                l = jnp.maximum(e - rc - rd - MARGIN, 0.0) + reg
            else:
                l = (MARGIN - e + rc + rd) + reg
            sl = pl.ds(pl.multiple_of(q * CH, CH) + pl.multiple_of(g * 16, 16), 16)
            if t == 0:
                ob[sl] = l
            else:
                ob[sl] = ob[sl] + l
            return carry

        lax.fori_loop(0, CH // 16, group, 0)

    cps = {0: fire(0), 1: fire(1)}
    rad_cp.wait()
    rel_cp.wait()
    for k in range(NT * NQ):
        for cp in cps[k]:
            cp.wait()
        compute(k)
        if k + 2 < NT * NQ:
            cps[k + 2] = fire(k + 2)

    def topg(g, carry):
        sl = pl.ds(pl.multiple_of(g * 16, 16), 16)
        tv = _rad_lookup(radtab, tix[sl])
        ob[sl] = ob[sl] + jnp.abs(tv - INF)
        return carry

    lax.fori_loop(0, BPW // 16, topg, 0)
    pltpu.sync_copy(ob, out_hbm.at[pl.ds(base, BPW)])


def _make_call():
    mesh = plsc.VectorSubcoreMesh(core_axis_name="c", subcore_axis_name="s",
                                  num_cores=NC, num_subcores=NS)
    return pl.kernel(
        _sc_body,
        out_type=jax.ShapeDtypeStruct((B,), jnp.float32),
        mesh=mesh,
        compiler_params=pltpu.CompilerParams(use_tc_tiling_on_sc=False,
                                             needs_layout_passes=False),
        scratch_types=[
            pltpu.VMEM((NRAD,), jnp.int32),
            pltpu.VMEM((NB_REL, HD), jnp.int32),
            pltpu.VMEM((CH, EMB), jnp.int32),
            pltpu.VMEM((CH, EMB), jnp.int32),
            pltpu.VMEM((CH, EMB), jnp.int32),
            pltpu.VMEM((CH, EMB), jnp.int32),
            pltpu.VMEM((CH,), jnp.int32),
            pltpu.VMEM((CH,), jnp.int32),
            pltpu.VMEM((CH,), jnp.int32),
            pltpu.VMEM((CH,), jnp.int32),
            pltpu.VMEM((NT * BPW,), jnp.int32),
            pltpu.VMEM((NT * BPW,), jnp.int32),
            pltpu.VMEM((NT * BPW,), jnp.int32),
            pltpu.VMEM((BPW,), jnp.int32),
            pltpu.VMEM((BPW,), jnp.float32),
            pltpu.SemaphoreType.DMA,
            pltpu.SemaphoreType.DMA,
            pltpu.SemaphoreType.DMA,
        ],
    )


def _arrange(cols):
    x = jnp.stack(cols, axis=0)
    x = x.reshape(NT, NW, BPW).swapaxes(0, 1)
    return x.reshape(-1)


def kernel(nf1, nf3, nf4, top, nf3_neg, cls_emb, rel_emb):
    ci = _arrange([nf1[:, 0], nf3[:, 0], nf4[:, 1], nf3_neg[:, 0]])
    di = _arrange([nf1[:, 2], nf3[:, 2], nf4[:, 2], nf3_neg[:, 2]])
    ri = _arrange([nf1[:, 1], nf3[:, 1], nf4[:, 0], nf3_neg[:, 1]])
    ti = top[:, 0]
    cls_p, rad_p = _prep_call(cls_emb.T)
    rel_p = _prep_rel_call(rel_emb.T)
    out = _make_call()(cls_p, rad_p, rel_p, ci, di, ri, ti)
    return out.reshape(B, 1)
